# baseline (speedup 1.0000x reference)
import functools

import jax
import jax.numpy as jnp
from jax import lax
from jax.experimental import pallas as pl
from jax.experimental.pallas import tpu as pltpu
from jax.experimental.pallas import tpu_sc as plsc

MAX_LEN = 2048
D_MODEL = 1024
BATCH = 4

_NC = 2
_ROWS_C = MAX_LEN // _NC


@functools.partial(
    pl.kernel,
    mesh=plsc.ScalarSubcoreMesh(axis_name="c", num_cores=_NC),
    out_type=jax.ShapeDtypeStruct((BATCH, MAX_LEN, D_MODEL), jnp.float32),
    scratch_types=[
        pltpu.VMEM_SHARED((_ROWS_C, D_MODEL), jnp.float32),
        pltpu.SemaphoreType.DMA,
        pltpu.SemaphoreType.DMA,
    ],
)
def _pe_broadcast(table_hbm, out_hbm, stage, sem_g, sem_s):
    cid = lax.axis_index("c")
    base = cid * _ROWS_C
    pltpu.async_copy(table_hbm.at[pl.ds(base, _ROWS_C), :], stage, sem_g).wait()
    copies = [
        pltpu.async_copy(stage, out_hbm.at[b, pl.ds(base, _ROWS_C), :], sem_s)
        for b in range(BATCH)
    ]
    for c in copies:
        c.wait()


def kernel(x, pe_table):
    del x
    return _pe_broadcast(pe_table)


# speedup vs baseline: 1.2766x; 1.2766x over previous
import functools

import jax
import jax.numpy as jnp
from jax import lax
from jax.experimental import pallas as pl
from jax.experimental.pallas import tpu as pltpu
from jax.experimental.pallas import tpu_sc as plsc

MAX_LEN = 2048
D_MODEL = 1024
BATCH = 4

_NC = 2
_NS = 16
_NW = _NC * _NS
_ROWS_W = MAX_LEN // _NW


@functools.partial(
    pl.kernel,
    mesh=plsc.VectorSubcoreMesh(core_axis_name="c", subcore_axis_name="s"),
    out_type=jax.ShapeDtypeStruct((BATCH, MAX_LEN, D_MODEL), jnp.float32),
    scratch_types=[
        pltpu.VMEM((_ROWS_W, D_MODEL), jnp.float32),
        pltpu.SemaphoreType.DMA,
        pltpu.SemaphoreType.DMA,
    ],
)
def _pe_broadcast(table_hbm, out_hbm, rows_v, sem_g, sem_s):
    wid = lax.axis_index("s") * _NC + lax.axis_index("c")
    base = wid * _ROWS_W
    pltpu.async_copy(table_hbm.at[pl.ds(base, _ROWS_W), :], rows_v, sem_g).wait()

    def body(b, carry):
        pltpu.async_copy(
            rows_v, out_hbm.at[b, pl.ds(base, _ROWS_W), :], sem_s
        ).wait()
        return carry

    lax.fori_loop(0, BATCH, body, 0)


def kernel(x, pe_table):
    del x
    return _pe_broadcast(pe_table)
